# baseline (speedup 1.0000x reference)
import jax
import jax.numpy as jnp
from jax import lax
from jax.experimental import pallas as pl
from jax.experimental.pallas import tpu as pltpu
from jax.experimental.pallas import tpu_sc as plsc

OUT_H = 7
OUT_W = 7
SPATIAL_SCALE = 0.25
SAMPLE_NUM = 2

NC = 2
NS = 16
NW = NC * NS

A = OUT_H * SAMPLE_NUM
NXV = 2 * A
GROWS = 4 * NXV


def _build(H, W, C, NPAD):
    RPW = NPAD // NW

    def body(f2_hbm, rois_hbm, out_hbm, roib, xvv,
             idx0, idx1, g0, g1, obuf, sem0, sem1):
        cid = lax.axis_index("c")
        sid = lax.axis_index("s")
        wid = sid * NC + cid
        pltpu.sync_copy(rois_hbm.at[pl.ds(wid * RPW, RPW)], roib)

        lane = lax.iota(jnp.int32, 16)
        pf = (lane >> 1).astype(jnp.float32)
        ff = ((lane & 1).astype(jnp.float32) + 0.5) * (1.0 / SAMPLE_NUM)
        lm = lane < A

        def roi_body(r, carry):
            rv = roib[r]
            x1 = rv[1] * SPATIAL_SCALE
            y1 = rv[2] * SPATIAL_SCALE
            x2 = rv[3] * SPATIAL_SCALE
            y2 = rv[4] * SPATIAL_SCALE
            bw = jnp.maximum(x2 - x1, 1.0) * (1.0 / OUT_W)
            bh = jnp.maximum(y2 - y1, 1.0) * (1.0 / OUT_H)

            posx = x1 + (pf + ff) * bw
            vx = jnp.where((posx >= -1.0) & (posx <= float(W)), 1.0, 0.0)
            xc = jnp.minimum(jnp.maximum(posx, 0.0), float(W - 1))
            xlo = xc.astype(jnp.int32)
            xhi = jnp.minimum(xlo + 1, W - 1)
            fx = xc - xlo.astype(jnp.float32)
            hxv = (1.0 - fx) * vx
            lxv = fx * vx
            plsc.store_scatter(xvv, [lane * 2], xlo, mask=lm)
            plsc.store_scatter(xvv, [lane * 2 + 1], xhi, mask=lm)

            posy = y1 + (pf + ff) * bh
            vy = jnp.where((posy >= -1.0) & (posy <= float(H)),
                           1.0 / (SAMPLE_NUM * SAMPLE_NUM), 0.0)
            yc = jnp.minimum(jnp.maximum(posy, 0.0), float(H - 1))
            ylo = yc.astype(jnp.int32)
            yhi = jnp.minimum(ylo + 1, H - 1)
            fy = yc - ylo.astype(jnp.float32)
            hyv = (1.0 - fy) * vy
            lyv = fy * vy
            ylo_s = ylo * W
            yhi_s = yhi * W

            xv_a = xvv[pl.ds(0, 16)]
            xv_b = xvv[pl.ds(16, 16)]

            def build_idx(ph, ib):
                ybases = (ylo_s[2 * ph], yhi_s[2 * ph],
                          ylo_s[2 * ph + 1], yhi_s[2 * ph + 1])
                for i in range(4):
                    plsc.store_scatter(ib, [lane + i * NXV], xv_a + ybases[i])
                    plsc.store_scatter(ib, [lane + (i * NXV + 16)],
                                       xv_b + ybases[i], mask=lane < NXV - 16)

            idxs = (idx0, idx1)
            gs = (g0, g1)
            sems = (sem0, sem1)
            build_idx(0, idxs[0])
            pend = [pltpu.async_copy(f2_hbm.at[idxs[0]], gs[0], sems[0]), None]
            for ph in range(OUT_H):
                cur = ph & 1
                if ph + 1 < OUT_H:
                    build_idx(ph + 1, idxs[1 - cur])
                    pend[1 - cur] = pltpu.async_copy(
                        f2_hbm.at[idxs[1 - cur]], gs[1 - cur], sems[1 - cur])
                pend[cur].wait()
                g = gs[cur]
                wyb = tuple(
                    plsc.pack(jnp.broadcast_to(w, (16,)),
                              jnp.broadcast_to(w, (16,)),
                              format=plsc.PackFormat.INTERLEAVED)
                    for w in (hyv[2 * ph], lyv[2 * ph],
                              hyv[2 * ph + 1], lyv[2 * ph + 1]))
                for pw in range(OUT_W):
                    wxb = tuple(
                        plsc.pack(jnp.broadcast_to(w, (16,)),
                                  jnp.broadcast_to(w, (16,)),
                                  format=plsc.PackFormat.INTERLEAVED)
                        for w in (hxv[2 * pw], lxv[2 * pw],
                                  hxv[2 * pw + 1], lxv[2 * pw + 1]))
                    orow = ph * OUT_W + pw

                    def cc_body(cc, z, g=g, wxb=wxb, wyb=wyb, pw=pw, orow=orow):
                        col = cc * 16
                        acc = None
                        for i in range(4):
                            t = None
                            for j in range(4):
                                gv = plsc.bitcast(
                                    g[i * NXV + 4 * pw + j, pl.ds(col, 16)],
                                    jnp.bfloat16)
                                t = gv * wxb[j] if t is None else t + gv * wxb[j]
                            acc = t * wyb[i] if acc is None else acc + t * wyb[i]
                        obuf[pl.ds(orow * (C // 2) + col, 16)] = plsc.bitcast(
                            acc, jnp.int32)
                        return z

                    lax.fori_loop(0, C // 32, cc_body, 0)
            pltpu.sync_copy(obuf, out_hbm.at[wid * RPW + r])
            return carry

        lax.fori_loop(0, RPW, roi_body, 0)

    return pl.kernel(
        body,
        out_type=jax.ShapeDtypeStruct((NPAD, OUT_H * OUT_W * C // 2),
                                      jnp.int32),
        mesh=plsc.VectorSubcoreMesh(core_axis_name="c", subcore_axis_name="s",
                                    num_cores=NC, num_subcores=NS),
        compiler_params=pltpu.CompilerParams(needs_layout_passes=False),
        scratch_types=[
            pltpu.VMEM((RPW, 16), jnp.float32),
            pltpu.VMEM((2 * 16,), jnp.int32),
            pltpu.VMEM((GROWS,), jnp.int32),
            pltpu.VMEM((GROWS,), jnp.int32),
            pltpu.VMEM((GROWS, C // 2), jnp.int32),
            pltpu.VMEM((GROWS, C // 2), jnp.int32),
            pltpu.VMEM((OUT_H * OUT_W * C // 2,), jnp.int32),
            pltpu.SemaphoreType.DMA,
            pltpu.SemaphoreType.DMA,
        ],
    )


def kernel(features, rois):
    B, C, H, W = features.shape
    N = rois.shape[0]
    fc = features[0].reshape(C, H * W)
    u = lax.bitcast_convert_type(fc, jnp.uint32)
    hi = (u + 0x7FFF + ((u >> 16) & 1)) >> 16
    packed = hi[0::2, :] | (hi[1::2, :] << 16)
    f2 = lax.bitcast_convert_type(packed, jnp.int32).T
    NPAD = ((N + NW - 1) // NW) * NW
    rois_p = jnp.pad(rois, ((0, NPAD - N), (0, 16 - rois.shape[1])))
    out = _build(H, W, C, NPAD)(f2, rois_p)
    out = lax.bitcast_convert_type(
        out[:N].reshape(N, OUT_H * OUT_W, C // 2), jnp.bfloat16)
    out = (out.reshape(N, OUT_H * OUT_W, C).transpose(0, 2, 1)
           .astype(jnp.float32).reshape(N, C, OUT_H, OUT_W))
    return out


# speedup vs baseline: 2.5100x; 2.5100x over previous
import jax
import jax.numpy as jnp
from jax import lax
from jax.experimental import pallas as pl
from jax.experimental.pallas import tpu as pltpu
from jax.experimental.pallas import tpu_sc as plsc

OUT_H = 7
OUT_W = 7
SPATIAL_SCALE = 0.25
SAMPLE_NUM = 2

NC = 2
NS = 16
NW = NC * NS

A = OUT_H * SAMPLE_NUM
NXV = 2 * A
GROWS = 4 * NXV


def _build(H, W, C, NPAD):
    RPW = NPAD // NW

    def body(f2_hbm, rois_hbm, out_hbm, roib, xvv,
             idx0, idx1, g0, g1, obuf, sem0, sem1):
        cid = lax.axis_index("c")
        sid = lax.axis_index("s")
        wid = sid * NC + cid
        pltpu.sync_copy(rois_hbm.at[pl.ds(wid * RPW, RPW)], roib)

        lane = lax.iota(jnp.int32, 16)
        pf = (lane >> 1).astype(jnp.float32)
        ff = ((lane & 1).astype(jnp.float32) + 0.5) * (1.0 / SAMPLE_NUM)
        lm = lane < A

        def roi_body(r, carry):
            rv = roib[r]
            x1 = rv[1] * SPATIAL_SCALE
            y1 = rv[2] * SPATIAL_SCALE
            x2 = rv[3] * SPATIAL_SCALE
            y2 = rv[4] * SPATIAL_SCALE
            bw = jnp.maximum(x2 - x1, 1.0) * (1.0 / OUT_W)
            bh = jnp.maximum(y2 - y1, 1.0) * (1.0 / OUT_H)

            posx = x1 + (pf + ff) * bw
            vx = jnp.where((posx >= -1.0) & (posx <= float(W)), 1.0, 0.0)
            xc = jnp.minimum(jnp.maximum(posx, 0.0), float(W - 1))
            xlo = xc.astype(jnp.int32)
            xhi = jnp.minimum(xlo + 1, W - 1)
            fx = xc - xlo.astype(jnp.float32)
            hxv = (1.0 - fx) * vx
            lxv = fx * vx
            plsc.store_scatter(xvv, [lane * 2], xlo, mask=lm)
            plsc.store_scatter(xvv, [lane * 2 + 1], xhi, mask=lm)

            posy = y1 + (pf + ff) * bh
            vy = jnp.where((posy >= -1.0) & (posy <= float(H)),
                           1.0 / (SAMPLE_NUM * SAMPLE_NUM), 0.0)
            yc = jnp.minimum(jnp.maximum(posy, 0.0), float(H - 1))
            ylo = yc.astype(jnp.int32)
            yhi = jnp.minimum(ylo + 1, H - 1)
            fy = yc - ylo.astype(jnp.float32)
            hyv = (1.0 - fy) * vy
            lyv = fy * vy
            ylo_s = ylo * W
            yhi_s = yhi * W

            xv_a = xvv[pl.ds(0, 16)]
            xv_b = xvv[pl.ds(16, 16)]

            def build_idx(ph, ib):
                ybases = (ylo_s[2 * ph], yhi_s[2 * ph],
                          ylo_s[2 * ph + 1], yhi_s[2 * ph + 1])
                for i in range(4):
                    plsc.store_scatter(ib, [lane + i * NXV], xv_a + ybases[i])
                    plsc.store_scatter(ib, [lane + (i * NXV + 16)],
                                       xv_b + ybases[i], mask=lane < NXV - 16)

            idxs = (idx0, idx1)
            gs = (g0, g1)
            sems = (sem0, sem1)
            build_idx(0, idxs[0])
            pend = [pltpu.async_copy(f2_hbm.at[idxs[0]], gs[0], sems[0]), None]
            for ph in range(OUT_H):
                cur = ph & 1
                if ph + 1 < OUT_H:
                    build_idx(ph + 1, idxs[1 - cur])
                    pend[1 - cur] = pltpu.async_copy(
                        f2_hbm.at[idxs[1 - cur]], gs[1 - cur], sems[1 - cur])
                pend[cur].wait()
                g = gs[cur]
                wyb = tuple(
                    plsc.pack(jnp.broadcast_to(w, (16,)),
                              jnp.broadcast_to(w, (16,)),
                              format=plsc.PackFormat.INTERLEAVED)
                    for w in (hyv[2 * ph], lyv[2 * ph],
                              hyv[2 * ph + 1], lyv[2 * ph + 1]))
                for pw in range(OUT_W):
                    wxb = tuple(
                        plsc.pack(jnp.broadcast_to(w, (16,)),
                                  jnp.broadcast_to(w, (16,)),
                                  format=plsc.PackFormat.INTERLEAVED)
                        for w in (hxv[2 * pw], lxv[2 * pw],
                                  hxv[2 * pw + 1], lxv[2 * pw + 1]))
                    orow = ph * OUT_W + pw

                    def cc_body(cc, z, g=g, wxb=wxb, wyb=wyb, pw=pw, orow=orow):
                        col = cc * 16
                        acc = None
                        for i in range(4):
                            t = None
                            for j in range(4):
                                gv = plsc.bitcast(
                                    g[i * NXV + 4 * pw + j, pl.ds(col, 16)],
                                    jnp.bfloat16)
                                t = gv * wxb[j] if t is None else t + gv * wxb[j]
                            acc = t * wyb[i] if acc is None else acc + t * wyb[i]
                        obuf[pl.ds(orow * (C // 2) + col, 16)] = plsc.bitcast(
                            acc, jnp.int32)
                        return z

                    lax.fori_loop(0, C // 32, cc_body, 0)
            pltpu.sync_copy(obuf, out_hbm.at[wid * RPW + r])
            return carry

        lax.fori_loop(0, RPW, roi_body, 0)

    return pl.kernel(
        body,
        out_type=jax.ShapeDtypeStruct((NPAD, OUT_H * OUT_W * C // 2),
                                      jnp.int32),
        mesh=plsc.VectorSubcoreMesh(core_axis_name="c", subcore_axis_name="s",
                                    num_cores=NC, num_subcores=NS),
        compiler_params=pltpu.CompilerParams(needs_layout_passes=False),
        scratch_types=[
            pltpu.VMEM((RPW, 16), jnp.float32),
            pltpu.VMEM((2 * 16,), jnp.int32),
            pltpu.VMEM((GROWS,), jnp.int32),
            pltpu.VMEM((GROWS,), jnp.int32),
            pltpu.VMEM((GROWS, C // 2), jnp.int32),
            pltpu.VMEM((GROWS, C // 2), jnp.int32),
            pltpu.VMEM((OUT_H * OUT_W * C // 2,), jnp.int32),
            pltpu.SemaphoreType.DMA,
            pltpu.SemaphoreType.DMA,
        ],
    )


def kernel(features, rois):
    B, C, H, W = features.shape
    N = rois.shape[0]
    b = features[0].reshape(C // 2, 2, H * W).astype(jnp.bfloat16)
    f2 = lax.bitcast_convert_type(b.transpose(2, 0, 1), jnp.int32)
    NPAD = ((N + NW - 1) // NW) * NW
    rois_p = jnp.pad(rois, ((0, NPAD - N), (0, 16 - rois.shape[1])))
    out = _build(H, W, C, NPAD)(f2, rois_p)
    out = lax.bitcast_convert_type(
        out[:N].reshape(N, OUT_H * OUT_W, C // 2), jnp.bfloat16)
    out = (out.reshape(N, OUT_H * OUT_W, C).transpose(0, 2, 1)
           .astype(jnp.float32).reshape(N, C, OUT_H, OUT_W))
    return out


# speedup vs baseline: 2.9453x; 1.1734x over previous
import jax
import jax.numpy as jnp
from jax import lax
from jax.experimental import pallas as pl
from jax.experimental.pallas import tpu as pltpu
from jax.experimental.pallas import tpu_sc as plsc

OUT_H = 7
OUT_W = 7
SPATIAL_SCALE = 0.25
SAMPLE_NUM = 2

NC = 2
NS = 16
NW = NC * NS

A = OUT_H * SAMPLE_NUM
NXV = 2 * A
GROWS = 4 * NXV


def _build(H, W, C, NPAD):
    RPW = NPAD // NW

    def body(f2_hbm, rois_hbm, out_hbm, roib, xvv,
             idx0, idx1, g0, g1, obuf, sem0, sem1):
        cid = lax.axis_index("c")
        sid = lax.axis_index("s")
        wid = sid * NC + cid
        pltpu.sync_copy(rois_hbm.at[pl.ds(wid * RPW, RPW)], roib)

        lane = lax.iota(jnp.int32, 16)
        pf = (lane >> 1).astype(jnp.float32)
        ff = ((lane & 1).astype(jnp.float32) + 0.5) * (1.0 / SAMPLE_NUM)
        lm = lane < A

        def roi_body(r, carry):
            rv = roib[r]
            x1 = rv[1] * SPATIAL_SCALE
            y1 = rv[2] * SPATIAL_SCALE
            x2 = rv[3] * SPATIAL_SCALE
            y2 = rv[4] * SPATIAL_SCALE
            bw = jnp.maximum(x2 - x1, 1.0) * (1.0 / OUT_W)
            bh = jnp.maximum(y2 - y1, 1.0) * (1.0 / OUT_H)

            posx = x1 + (pf + ff) * bw
            vx = jnp.where((posx >= -1.0) & (posx <= float(W)), 1.0, 0.0)
            xc = jnp.minimum(jnp.maximum(posx, 0.0), float(W - 1))
            xlo = xc.astype(jnp.int32)
            xhi = jnp.minimum(xlo + 1, W - 1)
            fx = xc - xlo.astype(jnp.float32)
            hxv = (1.0 - fx) * vx
            lxv = fx * vx
            plsc.store_scatter(xvv, [lane * 2], xlo, mask=lm)
            plsc.store_scatter(xvv, [lane * 2 + 1], xhi, mask=lm)

            posy = y1 + (pf + ff) * bh
            vy = jnp.where((posy >= -1.0) & (posy <= float(H)),
                           1.0 / (SAMPLE_NUM * SAMPLE_NUM), 0.0)
            yc = jnp.minimum(jnp.maximum(posy, 0.0), float(H - 1))
            ylo = yc.astype(jnp.int32)
            yhi = jnp.minimum(ylo + 1, H - 1)
            fy = yc - ylo.astype(jnp.float32)
            hyv = (1.0 - fy) * vy
            lyv = fy * vy
            ylo_s = ylo * W
            yhi_s = yhi * W

            xv_a = xvv[pl.ds(0, 16)]
            xv_b = xvv[pl.ds(16, 16)]

            def build_idx(ph, ib):
                ybases = (ylo_s[2 * ph], yhi_s[2 * ph],
                          ylo_s[2 * ph + 1], yhi_s[2 * ph + 1])
                for i in range(4):
                    plsc.store_scatter(ib, [lane + i * NXV], xv_a + ybases[i])
                    plsc.store_scatter(ib, [lane + (i * NXV + 16)],
                                       xv_b + ybases[i], mask=lane < NXV - 16)

            idxs = (idx0, idx1)
            gs = (g0, g1)
            sems = (sem0, sem1)
            build_idx(0, idxs[0])
            pend = [pltpu.async_copy(f2_hbm.at[idxs[0]], gs[0], sems[0]), None]
            for ph in range(OUT_H):
                cur = ph & 1
                if ph + 1 < OUT_H:
                    build_idx(ph + 1, idxs[1 - cur])
                    pend[1 - cur] = pltpu.async_copy(
                        f2_hbm.at[idxs[1 - cur]], gs[1 - cur], sems[1 - cur])
                pend[cur].wait()
                g = gs[cur]
                wyb = tuple(
                    plsc.pack(jnp.broadcast_to(w, (16,)),
                              jnp.broadcast_to(w, (16,)),
                              format=plsc.PackFormat.INTERLEAVED)
                    for w in (hyv[2 * ph], lyv[2 * ph],
                              hyv[2 * ph + 1], lyv[2 * ph + 1]))
                for pw in range(OUT_W):
                    wxb = tuple(
                        plsc.pack(jnp.broadcast_to(w, (16,)),
                                  jnp.broadcast_to(w, (16,)),
                                  format=plsc.PackFormat.INTERLEAVED)
                        for w in (hxv[2 * pw], lxv[2 * pw],
                                  hxv[2 * pw + 1], lxv[2 * pw + 1]))
                    orow = ph * OUT_W + pw

                    def cc_body(cc, z, g=g, wxb=wxb, wyb=wyb, pw=pw, orow=orow):
                        col = cc * 16
                        acc = None
                        for i in range(4):
                            t = None
                            for j in range(4):
                                gv = plsc.bitcast(
                                    g[i * NXV + 4 * pw + j, pl.ds(col, 16)],
                                    jnp.bfloat16)
                                t = gv * wxb[j] if t is None else t + gv * wxb[j]
                            acc = t * wyb[i] if acc is None else acc + t * wyb[i]
                        obuf[pl.ds(orow * (C // 2) + col, 16)] = plsc.bitcast(
                            acc, jnp.int32)
                        return z

                    lax.fori_loop(0, C // 32, cc_body, 0)
            pltpu.sync_copy(obuf, out_hbm.at[wid * RPW + r])
            return carry

        lax.fori_loop(0, RPW, roi_body, 0)

    return pl.kernel(
        body,
        out_type=jax.ShapeDtypeStruct((NPAD, OUT_H * OUT_W * C // 2),
                                      jnp.int32),
        mesh=plsc.VectorSubcoreMesh(core_axis_name="c", subcore_axis_name="s",
                                    num_cores=NC, num_subcores=NS),
        compiler_params=pltpu.CompilerParams(needs_layout_passes=False),
        scratch_types=[
            pltpu.VMEM((RPW, 16), jnp.float32),
            pltpu.VMEM((2 * 16,), jnp.int32),
            pltpu.VMEM((GROWS,), jnp.int32),
            pltpu.VMEM((GROWS,), jnp.int32),
            pltpu.VMEM((GROWS, C // 2), jnp.int32),
            pltpu.VMEM((GROWS, C // 2), jnp.int32),
            pltpu.VMEM((OUT_H * OUT_W * C // 2,), jnp.int32),
            pltpu.SemaphoreType.DMA,
            pltpu.SemaphoreType.DMA,
        ],
    )


def kernel(features, rois):
    B, C, H, W = features.shape
    N = rois.shape[0]
    b2 = features[0].reshape(C // 2, 2, H * W).astype(jnp.bfloat16)
    lo = lax.bitcast_convert_type(b2[:, 0, :], jnp.uint16).astype(jnp.uint32)
    hi = lax.bitcast_convert_type(b2[:, 1, :], jnp.uint16).astype(jnp.uint32)
    f2 = lax.bitcast_convert_type(lo | (hi << 16), jnp.int32).T
    NPAD = ((N + NW - 1) // NW) * NW
    rois_p = jnp.pad(rois, ((0, NPAD - N), (0, 16 - rois.shape[1])))
    out = _build(H, W, C, NPAD)(f2, rois_p)
    out = lax.bitcast_convert_type(
        out[:N].reshape(N, OUT_H * OUT_W, C // 2), jnp.bfloat16)
    out = (out.reshape(N, OUT_H * OUT_W, C).transpose(0, 2, 1)
           .astype(jnp.float32).reshape(N, C, OUT_H, OUT_W))
    return out


# speedup vs baseline: 6.5754x; 2.2325x over previous
import jax
import jax.numpy as jnp
from jax import lax
from jax.experimental import pallas as pl
from jax.experimental.pallas import tpu as pltpu
from jax.experimental.pallas import tpu_sc as plsc

OUT_H = 7
OUT_W = 7
SPATIAL_SCALE = 0.25
SAMPLE_NUM = 2

NC = 2
NS = 16
NW = NC * NS

A = OUT_H * SAMPLE_NUM
NXV = 2 * A
GROWS = 4 * NXV


def _build(H, W, C, NPAD):
    RPW = NPAD // NW

    def body(f2_hbm, rois_hbm, out_hbm, roib, xvv,
             idx0, idx1, g0, g1, obuf, sem0, sem1):
        cid = lax.axis_index("c")
        sid = lax.axis_index("s")
        wid = sid * NC + cid
        pltpu.sync_copy(rois_hbm.at[pl.ds(wid * RPW, RPW)], roib)

        lane = lax.iota(jnp.int32, 16)
        pf = (lane >> 1).astype(jnp.float32)
        ff = ((lane & 1).astype(jnp.float32) + 0.5) * (1.0 / SAMPLE_NUM)
        lm = lane < A

        idxs = (idx0, idx1)
        gs = (g0, g1)
        sems = (sem0, sem1)

        def prologue(r):
            rv = roib[r]
            x1 = rv[1] * SPATIAL_SCALE
            y1 = rv[2] * SPATIAL_SCALE
            x2 = rv[3] * SPATIAL_SCALE
            y2 = rv[4] * SPATIAL_SCALE
            bw = jnp.maximum(x2 - x1, 1.0) * (1.0 / OUT_W)
            bh = jnp.maximum(y2 - y1, 1.0) * (1.0 / OUT_H)

            posx = x1 + (pf + ff) * bw
            vx = jnp.where((posx >= -1.0) & (posx <= float(W)), 1.0, 0.0)
            xc = jnp.minimum(jnp.maximum(posx, 0.0), float(W - 1))
            xlo = xc.astype(jnp.int32)
            xhi = jnp.minimum(xlo + 1, W - 1)
            fx = xc - xlo.astype(jnp.float32)
            hxv = (1.0 - fx) * vx
            lxv = fx * vx
            plsc.store_scatter(xvv, [lane * 2], xlo, mask=lm)
            plsc.store_scatter(xvv, [lane * 2 + 1], xhi, mask=lm)

            posy = y1 + (pf + ff) * bh
            vy = jnp.where((posy >= -1.0) & (posy <= float(H)),
                           1.0 / (SAMPLE_NUM * SAMPLE_NUM), 0.0)
            yc = jnp.minimum(jnp.maximum(posy, 0.0), float(H - 1))
            ylo = yc.astype(jnp.int32)
            yhi = jnp.minimum(ylo + 1, H - 1)
            fy = yc - ylo.astype(jnp.float32)
            hyv = (1.0 - fy) * vy
            lyv = fy * vy
            return (hxv, lxv, hyv, lyv, ylo * W, yhi * W)

        def build_idx(vecs, ph, ib):
            ylo_s, yhi_s = vecs[4], vecs[5]
            ybases = (ylo_s[2 * ph], yhi_s[2 * ph],
                      ylo_s[2 * ph + 1], yhi_s[2 * ph + 1])
            xv_a = xvv[pl.ds(0, 16)]
            xv_b = xvv[pl.ds(16, 16)]
            for i in range(4):
                plsc.store_scatter(ib, [lane + i * NXV], xv_a + ybases[i])
                plsc.store_scatter(ib, [lane + (i * NXV + 16)],
                                   xv_b + ybases[i], mask=lane < NXV - 16)

        def issue(b):
            return pltpu.async_copy(f2_hbm.at[idxs[b]], gs[b], sems[b])

        def process(vecs, r, par, prefetch):
            hxv, lxv, hyv, lyv = vecs[0], vecs[1], vecs[2], vecs[3]
            nvecs = None
            for ph in range(OUT_H):
                cur = (ph + par) & 1
                if ph + 1 < OUT_H:
                    build_idx(vecs, ph + 1, idxs[1 - cur])
                    issue(1 - cur)
                else:
                    nvecs = prefetch(1 - cur)
                pltpu.make_async_copy(f2_hbm.at[idxs[cur]], gs[cur],
                                      sems[cur]).wait()
                g = gs[cur]
                wy = (hyv[2 * ph], lyv[2 * ph],
                      hyv[2 * ph + 1], lyv[2 * ph + 1])
                for pw in range(OUT_W):
                    wx = (hxv[2 * pw], lxv[2 * pw],
                          hxv[2 * pw + 1], lxv[2 * pw + 1])
                    orow = ph * OUT_W + pw

                    def cc_body(cc, z, g=g, wx=wx, wy=wy, pw=pw, orow=orow):
                        col = cc * 16
                        acc = None
                        for i in range(4):
                            t = None
                            for j in range(4):
                                gv = g[i * NXV + 4 * pw + j, pl.ds(col, 16)]
                                t = gv * wx[j] if t is None else t + gv * wx[j]
                            acc = t * wy[i] if acc is None else acc + t * wy[i]
                        obuf[orow, pl.ds(col, 16)] = acc
                        return z

                    lax.fori_loop(0, C // 16, cc_body, 0)
            return nvecs

        def pair_body(it, vecs_a):
            ra = 2 * it
            rb = 2 * it + 1

            def prefetch_b(buf):
                vecs_b = prologue(rb)
                build_idx(vecs_b, 0, idxs[buf])
                issue(buf)
                return vecs_b

            vecs_b = process(vecs_a, ra, 0, prefetch_b)
            pltpu.sync_copy(obuf, out_hbm.at[wid * RPW + ra])

            def prefetch_a2(buf):
                ra2 = jnp.minimum(2 * it + 2, RPW - 1)
                vecs_a2 = prologue(ra2)
                build_idx(vecs_a2, 0, idxs[buf])
                issue(buf)
                return vecs_a2

            vecs_a2 = process(vecs_b, rb, 1, prefetch_a2)
            pltpu.sync_copy(obuf, out_hbm.at[wid * RPW + rb])
            return vecs_a2

        vecs0 = prologue(0)
        build_idx(vecs0, 0, idxs[0])
        issue(0)
        lax.fori_loop(0, RPW // 2, pair_body, vecs0)
        pltpu.make_async_copy(f2_hbm.at[idxs[0]], gs[0], sems[0]).wait()

    return pl.kernel(
        body,
        out_type=jax.ShapeDtypeStruct((NPAD, OUT_H * OUT_W, C), jnp.float32),
        mesh=plsc.VectorSubcoreMesh(core_axis_name="c", subcore_axis_name="s",
                                    num_cores=NC, num_subcores=NS),
        compiler_params=pltpu.CompilerParams(needs_layout_passes=False),
        scratch_types=[
            pltpu.VMEM((RPW, 16), jnp.float32),
            pltpu.VMEM((2 * 16,), jnp.int32),
            pltpu.VMEM((GROWS,), jnp.int32),
            pltpu.VMEM((GROWS,), jnp.int32),
            pltpu.VMEM((GROWS, C), jnp.float32),
            pltpu.VMEM((GROWS, C), jnp.float32),
            pltpu.VMEM((OUT_H * OUT_W, C), jnp.float32),
            pltpu.SemaphoreType.DMA,
            pltpu.SemaphoreType.DMA,
        ],
    )


def kernel(features, rois):
    B, C, H, W = features.shape
    N = rois.shape[0]
    f2 = features[0].transpose(1, 2, 0).reshape(H * W, C)
    NPAD = ((N + 2 * NW - 1) // (2 * NW)) * (2 * NW)
    rois_p = jnp.pad(rois, ((0, NPAD - N), (0, 16 - rois.shape[1])))
    out = _build(H, W, C, NPAD)(f2, rois_p)
    out = out[:N].reshape(N, OUT_H, OUT_W, C).transpose(0, 3, 1, 2)
    return out
